# Initial kernel scaffold; baseline (speedup 1.0000x reference)
import functools

import jax
import jax.numpy as jnp
from jax import lax
from jax.experimental import pallas as pl
from jax.experimental.pallas import tpu as pltpu
from jax.experimental.pallas import tpu_sc as plsc

NV = 50000
NN = 51200
PT = 3200
W = 800
E = 800000
ECH = 128
NCHUNK = 392
EPAD = 16 * NCHUNK * ECH
TRASH = NV


def _body(rowp, colp, embp, z2, z1, outf, gsrc, macc,
          acc_sh, deg_sh, zbuf, rv, wv, sv, dvv, rowv, colv, onesv, sem):
    c = lax.axis_index("c")
    s = lax.axis_index("s")
    coff = c * NN

    pltpu.sync_copy(z2, zbuf)
    for t in range(8):
        onesv[0, pl.ds(16 * t, 16)] = jnp.full((16,), 1.0, jnp.float32)
    pltpu.sync_copy(z1, deg_sh.at[pl.ds(s * PT, PT)])
    plsc.subcore_barrier()

    def deg_step(j, _):
        pltpu.sync_copy(rowp.at[s, j], rowv.at[0])
        pltpu.sync_copy(onesv.at[0], deg_sh.at[rowv.at[0]], add=True)
        pltpu.sync_copy(colp.at[s, j], colv.at[0])
        pltpu.sync_copy(onesv.at[0], deg_sh.at[colv.at[0]], add=True)
        return 0

    lax.fori_loop(0, NCHUNK, deg_step, 0)
    plsc.subcore_barrier()

    for k in range(4):
        base = s * PT + k * W
        gb = coff + base
        pltpu.sync_copy(deg_sh.at[pl.ds(base, W)], dvv)

        def rsqrt_step(g, _):
            x = jnp.maximum(dvv[pl.ds(g * 16, 16)], 1.0)
            bits = lax.bitcast_convert_type(x, jnp.int32)
            y = lax.bitcast_convert_type(
                jnp.int32(0x5F3759DF) - lax.shift_right_arithmetic(bits, 1),
                jnp.float32)
            half = x * 0.5
            for _ in range(3):
                y = y * (1.5 - half * y * y)
            dvv[pl.ds(g * 16, 16)] = y
            return 0

        lax.fori_loop(0, W // 16, rsqrt_step, 0)
        pltpu.sync_copy(dvv, deg_sh.at[pl.ds(base, W)])
        pltpu.sync_copy(embp.at[pl.ds(base, W), pl.ds(c * 32, 32)], wv)
        pltpu.sync_copy(wv, macc.at[pl.ds(gb, W), :])

        def scale_step(g, _):
            for i in range(16):
                n = g * 16 + i
                d = jnp.full((16,), dvv[n])
                wv[n, pl.ds(0, 16)] = wv[n, pl.ds(0, 16)] * d
                wv[n, pl.ds(16, 16)] = wv[n, pl.ds(16, 16)] * d
            return 0

        lax.fori_loop(0, W // 16, scale_step, 0)
        pltpu.sync_copy(wv, gsrc.at[pl.ds(gb, W), :])
    plsc.subcore_barrier()

    for layer in range(3):
        last = layer == 2
        for k in range(4):
            pltpu.sync_copy(zbuf, acc_sh.at[pl.ds(s * PT + k * W, W), :])
        plsc.subcore_barrier()

        def edge_step(j, _):
            pltpu.sync_copy(colp.at[s, j], colv.at[0])
            for t in range(8):
                colv[0, pl.ds(16 * t, 16)] = (
                    colv[0, pl.ds(16 * t, 16)] + jnp.full((16,), coff))
            pltpu.async_copy(gsrc.at[colv.at[0]], rv, sem).wait()
            pltpu.sync_copy(rowp.at[s, j], rowv.at[0])
            pltpu.sync_copy(rv, acc_sh.at[rowv.at[0]], add=True)
            return 0

        lax.fori_loop(0, NCHUNK, edge_step, 0)
        plsc.subcore_barrier()

        for k in range(4):
            base = s * PT + k * W
            gb = coff + base
            pltpu.sync_copy(acc_sh.at[pl.ds(base, W), :], sv)
            pltpu.sync_copy(deg_sh.at[pl.ds(base, W)], dvv)
            pltpu.sync_copy(macc.at[pl.ds(gb, W), :], wv)

            def out_step(g, _):
                for i in range(16):
                    n = g * 16 + i
                    d = jnp.full((16,), dvv[n])
                    e0 = d * sv[n, pl.ds(0, 16)]
                    e1 = d * sv[n, pl.ds(16, 16)]
                    m0 = wv[n, pl.ds(0, 16)] + e0
                    m1 = wv[n, pl.ds(16, 16)] + e1
                    if last:
                        wv[n, pl.ds(0, 16)] = m0 * 0.25
                        wv[n, pl.ds(16, 16)] = m1 * 0.25
                    else:
                        wv[n, pl.ds(0, 16)] = m0
                        wv[n, pl.ds(16, 16)] = m1
                        sv[n, pl.ds(0, 16)] = d * e0
                        sv[n, pl.ds(16, 16)] = d * e1
                return 0

            lax.fori_loop(0, W // 16, out_step, 0)
            if last:
                pltpu.sync_copy(wv, outf.at[pl.ds(gb, W), :])
            else:
                pltpu.sync_copy(wv, macc.at[pl.ds(gb, W), :])
                pltpu.sync_copy(sv, gsrc.at[pl.ds(gb, W), :])
        plsc.subcore_barrier()


_mesh = plsc.VectorSubcoreMesh(core_axis_name="c", subcore_axis_name="s")

_sc_call = pl.kernel(
    _body,
    out_type=(
        jax.ShapeDtypeStruct((2 * NN, 32), jnp.float32),
        jax.ShapeDtypeStruct((2 * NN, 32), jnp.float32),
        jax.ShapeDtypeStruct((2 * NN, 32), jnp.float32),
    ),
    mesh=_mesh,
    scratch_types=[
        pltpu.VMEM_SHARED((NN, 32), jnp.float32),
        pltpu.VMEM_SHARED((NN,), jnp.float32),
        pltpu.VMEM((W, 32), jnp.float32),
        pltpu.VMEM((ECH, 32), jnp.float32),
        pltpu.VMEM((W, 32), jnp.float32),
        pltpu.VMEM((W, 32), jnp.float32),
        pltpu.VMEM((W,), jnp.float32),
        pltpu.VMEM((1, ECH), jnp.int32),
        pltpu.VMEM((1, ECH), jnp.int32),
        pltpu.VMEM((1, ECH), jnp.float32),
        pltpu.SemaphoreType.DMA,
    ],
)


@jax.jit
def kernel(edge_index, embedding_weight):
    row = edge_index[0]
    col = edge_index[1]
    pad = jnp.full((EPAD - E,), TRASH, jnp.int32)
    rowp = jnp.concatenate([row, pad]).reshape(16, NCHUNK, ECH)
    colp = jnp.concatenate([col, pad]).reshape(16, NCHUNK, ECH)
    embp = jnp.zeros((NN, 64), jnp.float32).at[:NV].set(embedding_weight)
    z2 = jnp.zeros((W, 32), jnp.float32)
    z1 = jnp.zeros((PT,), jnp.float32)
    outf, _, _ = _sc_call(rowp, colp, embp, z2, z1)
    final = jnp.concatenate([outf[:NV], outf[NN:NN + NV]], axis=1)
    return final[:NV // 2], final[NV // 2:]


# speedup vs baseline: 7.3778x; 7.3778x over previous
import jax
import jax.numpy as jnp
from jax import lax
from jax.experimental import pallas as pl
from jax.experimental.pallas import tpu as pltpu
from jax.experimental.pallas import tpu_sc as plsc

NV = 50000
NN = 51200
PT = 3200
W = 320
NK = PT // W
E = 800000
ECH = 128
NCHUNK = 392
EPAD = 16 * NCHUNK * ECH
TRASH = NV


def _body(rowp, colp, embp, z2, z1, outf, gsrc, macc,
          acc_sh, deg_sh, wv, sv, dvv, rowv, colv, onesv, sem):
    c = lax.axis_index("c")
    s = lax.axis_index("s")
    coff = c * NN

    for t in range(8):
        onesv[0, pl.ds(16 * t, 16)] = jnp.full((16,), 1.0, jnp.float32)
    pltpu.sync_copy(z1, deg_sh.at[pl.ds(s * PT, PT)])
    plsc.subcore_barrier()

    def deg_step(j, _):
        pltpu.sync_copy(rowp.at[s, j], rowv.at[0])
        pltpu.sync_copy(onesv.at[0], deg_sh.at[rowv.at[0]], add=True)
        pltpu.sync_copy(colp.at[s, j], colv.at[0])
        pltpu.sync_copy(onesv.at[0], deg_sh.at[colv.at[0]], add=True)
        return 0

    lax.fori_loop(0, NCHUNK, deg_step, 0)
    plsc.subcore_barrier()

    def init_step(k, _):
        base = s * PT + k * W
        gb = coff + base
        pltpu.sync_copy(deg_sh.at[pl.ds(base, W)], dvv)

        def rsqrt_step(g, _):
            x = jnp.maximum(dvv[pl.ds(g * 16, 16)], 1.0)
            bits = lax.bitcast_convert_type(x, jnp.int32)
            y = lax.bitcast_convert_type(
                jnp.int32(0x5F3759DF) - lax.shift_right_arithmetic(bits, 1),
                jnp.float32)
            half = x * 0.5
            for _ in range(3):
                y = y * (1.5 - half * y * y)
            dvv[pl.ds(g * 16, 16)] = y
            return 0

        lax.fori_loop(0, W // 16, rsqrt_step, 0)
        pltpu.sync_copy(dvv, deg_sh.at[pl.ds(base, W)])
        pltpu.sync_copy(embp.at[pl.ds(gb, W), :], wv)
        pltpu.sync_copy(wv, macc.at[pl.ds(gb, W), :])

        def scale_step(g, _):
            d16 = dvv[pl.ds(g * 16, 16)]
            for i in range(16):
                n = g * 16 + i
                d = jnp.full((16,), d16[i])
                wv[n, pl.ds(0, 16)] = wv[n, pl.ds(0, 16)] * d
                wv[n, pl.ds(16, 16)] = wv[n, pl.ds(16, 16)] * d
            return 0

        lax.fori_loop(0, W // 16, scale_step, 0)
        pltpu.sync_copy(wv, gsrc.at[pl.ds(gb, W), :])
        return 0

    lax.fori_loop(0, NK, init_step, 0)
    plsc.subcore_barrier()

    for layer in range(3):
        last = layer == 2

        def zero_step(k, _):
            pltpu.sync_copy(z2, acc_sh.at[pl.ds(s * PT + k * W, W), :])
            return 0

        lax.fori_loop(0, NK, zero_step, 0)
        plsc.subcore_barrier()

        def edge_step(j, _):
            pltpu.sync_copy(colp.at[s, j], colv.at[0])
            for t in range(8):
                colv[0, pl.ds(16 * t, 16)] = (
                    colv[0, pl.ds(16 * t, 16)] + jnp.full((16,), coff))
            rv = wv.at[pl.ds(0, ECH), :]
            pltpu.async_copy(gsrc.at[colv.at[0]], rv, sem).wait()
            pltpu.sync_copy(rowp.at[s, j], rowv.at[0])
            pltpu.sync_copy(rv, acc_sh.at[rowv.at[0]], add=True)
            return 0

        lax.fori_loop(0, NCHUNK, edge_step, 0)
        plsc.subcore_barrier()

        def write_step(k, _):
            base = s * PT + k * W
            gb = coff + base
            pltpu.sync_copy(acc_sh.at[pl.ds(base, W), :], sv)
            pltpu.sync_copy(deg_sh.at[pl.ds(base, W)], dvv)
            pltpu.sync_copy(macc.at[pl.ds(gb, W), :], wv)

            def out_step(g, _):
                d16 = dvv[pl.ds(g * 16, 16)]
                for i in range(16):
                    n = g * 16 + i
                    d = jnp.full((16,), d16[i])
                    e0 = d * sv[n, pl.ds(0, 16)]
                    e1 = d * sv[n, pl.ds(16, 16)]
                    m0 = wv[n, pl.ds(0, 16)] + e0
                    m1 = wv[n, pl.ds(16, 16)] + e1
                    if last:
                        wv[n, pl.ds(0, 16)] = m0 * 0.25
                        wv[n, pl.ds(16, 16)] = m1 * 0.25
                    else:
                        wv[n, pl.ds(0, 16)] = m0
                        wv[n, pl.ds(16, 16)] = m1
                        sv[n, pl.ds(0, 16)] = d * e0
                        sv[n, pl.ds(16, 16)] = d * e1
                return 0

            lax.fori_loop(0, W // 16, out_step, 0)
            if last:
                pltpu.sync_copy(wv, outf.at[pl.ds(gb, W), :])
            else:
                pltpu.sync_copy(wv, macc.at[pl.ds(gb, W), :])
                pltpu.sync_copy(sv, gsrc.at[pl.ds(gb, W), :])
            return 0

        lax.fori_loop(0, NK, write_step, 0)
        plsc.subcore_barrier()


_mesh = plsc.VectorSubcoreMesh(core_axis_name="c", subcore_axis_name="s")

_sc_call = pl.kernel(
    _body,
    out_type=(
        jax.ShapeDtypeStruct((2 * NN, 32), jnp.float32),
        jax.ShapeDtypeStruct((2 * NN, 32), jnp.float32),
        jax.ShapeDtypeStruct((2 * NN, 32), jnp.float32),
    ),
    mesh=_mesh,
    compiler_params=pltpu.CompilerParams(use_tc_tiling_on_sc=False),
    scratch_types=[
        pltpu.VMEM_SHARED((NN, 32), jnp.float32),
        pltpu.VMEM_SHARED((NN,), jnp.float32),
        pltpu.VMEM((W, 32), jnp.float32),
        pltpu.VMEM((W, 32), jnp.float32),
        pltpu.VMEM((W,), jnp.float32),
        pltpu.VMEM((1, ECH), jnp.int32),
        pltpu.VMEM((1, ECH), jnp.int32),
        pltpu.VMEM((1, ECH), jnp.float32),
        pltpu.SemaphoreType.DMA,
    ],
)


@jax.jit
def kernel(edge_index, embedding_weight):
    row = edge_index[0]
    col = edge_index[1]
    pad = jnp.full((EPAD - E,), TRASH, jnp.int32)
    rowp = jnp.concatenate([row, pad]).reshape(16, NCHUNK, ECH)
    colp = jnp.concatenate([col, pad]).reshape(16, NCHUNK, ECH)
    embp = (jnp.zeros((2 * NN, 32), jnp.float32)
            .at[:NV].set(embedding_weight[:, :32])
            .at[NN:NN + NV].set(embedding_weight[:, 32:]))
    z2 = jnp.zeros((W, 32), jnp.float32)
    z1 = jnp.zeros((PT,), jnp.float32)
    outf, _, _ = _sc_call(rowp, colp, embp, z2, z1)
    final = jnp.concatenate([outf[:NV], outf[NN:NN + NV]], axis=1)
    return final[:NV // 2], final[NV // 2:]


# speedup vs baseline: 20.1960x; 2.7374x over previous
import jax
import jax.numpy as jnp
from jax import lax
from jax.experimental import pallas as pl
from jax.experimental.pallas import tpu as pltpu
from jax.experimental.pallas import tpu_sc as plsc

NV = 50000
NN = 50176
PT = 3136
W = 224
NK = PT // W
E = 800000
ECH = 128
F = 4
NB = 98
NCHUNK = NB * F
EPAD = 16 * NCHUNK * ECH
TRASH = NV


def _body(rowp, colp, embp, z2, z1, outf, gsrc, macc,
          acc_sh, deg_sh, wv, rvN, dvv, rowv, colv, onesv,
          isem, gsem, ssem):
    c = lax.axis_index("c")
    s = lax.axis_index("s")
    coff = c * NN

    for t in range(8):
        onesv[0, pl.ds(16 * t, 16)] = jnp.full((16,), 1.0, jnp.float32)
    pltpu.sync_copy(z1, deg_sh.at[pl.ds(s * PT, PT)])
    plsc.subcore_barrier()

    pltpu.sync_copy(rowp.at[s, 0], rowv.at[pl.ds(0, F), :])
    pltpu.sync_copy(colp.at[0, s, 0], colv.at[pl.ds(0, F), :])

    def deg_step(g, _):
        b = lax.rem(g, 2) * F
        nb = F - b
        di1 = pltpu.async_copy(rowp.at[s, g + 1],
                               rowv.at[pl.ds(nb, F), :], isem)
        di2 = pltpu.async_copy(colp.at[0, s, g + 1],
                               colv.at[pl.ds(nb, F), :], isem)
        sd = []
        for f in range(F):
            sd.append(pltpu.async_copy(
                onesv.at[0], deg_sh.at[rowv.at[b + f]], ssem, add=True))
            sd.append(pltpu.async_copy(
                onesv.at[0], deg_sh.at[colv.at[b + f]], ssem, add=True))
        for d in sd:
            d.wait()
        di1.wait()
        di2.wait()
        return 0

    lax.fori_loop(0, NB, deg_step, 0)
    plsc.subcore_barrier()

    def init_step(k, _):
        base = s * PT + k * W
        gb = coff + base
        pltpu.sync_copy(deg_sh.at[pl.ds(base, W)], dvv)

        def rsqrt_step(g, _):
            x = jnp.maximum(dvv[pl.ds(g * 16, 16)], 1.0)
            bits = lax.bitcast_convert_type(x, jnp.int32)
            y = lax.bitcast_convert_type(
                jnp.int32(0x5F3759DF) - lax.shift_right_arithmetic(bits, 1),
                jnp.float32)
            half = x * 0.5
            for _ in range(3):
                y = y * (1.5 - half * y * y)
            dvv[pl.ds(g * 16, 16)] = y
            return 0

        lax.fori_loop(0, W // 16, rsqrt_step, 0)
        pltpu.sync_copy(dvv, deg_sh.at[pl.ds(base, W)])
        pltpu.sync_copy(embp.at[pl.ds(gb, W), :], wv)
        pltpu.sync_copy(wv, macc.at[pl.ds(gb, W), :])

        def scale_step(g, _):
            d16 = dvv[pl.ds(g * 16, 16)]
            for i in range(16):
                n = g * 16 + i
                d = jnp.full((16,), d16[i])
                wv[n, pl.ds(0, 16)] = wv[n, pl.ds(0, 16)] * d
                wv[n, pl.ds(16, 16)] = wv[n, pl.ds(16, 16)] * d
            return 0

        lax.fori_loop(0, W // 16, scale_step, 0)
        pltpu.sync_copy(wv, gsrc.at[pl.ds(gb, W), :])
        return 0

    lax.fori_loop(0, NK, init_step, 0)
    plsc.subcore_barrier()

    for layer in range(3):
        last = layer == 2

        def zero_step(k, _):
            pltpu.sync_copy(z2, acc_sh.at[pl.ds(s * PT + k * W, W), :])
            return 0

        lax.fori_loop(0, NK, zero_step, 0)
        plsc.subcore_barrier()

        pltpu.sync_copy(rowp.at[s, 0], rowv.at[pl.ds(0, F), :])
        pltpu.sync_copy(colp.at[c, s, 0], colv.at[pl.ds(0, F), :])

        def edge_step(g, _):
            b = lax.rem(g, 2) * F
            nb = F - b
            di1 = pltpu.async_copy(rowp.at[s, g + 1],
                                   rowv.at[pl.ds(nb, F), :], isem)
            di2 = pltpu.async_copy(colp.at[c, s, g + 1],
                                   colv.at[pl.ds(nb, F), :], isem)
            gd = []
            for f in range(F):
                gd.append(pltpu.async_copy(
                    gsrc.at[colv.at[b + f]],
                    rvN.at[pl.ds(f * ECH, ECH), :], gsem))
            sd = []
            for f in range(F):
                gd[f].wait()
                sd.append(pltpu.async_copy(
                    rvN.at[pl.ds(f * ECH, ECH), :],
                    acc_sh.at[rowv.at[b + f]], ssem, add=True))
            for d in sd:
                d.wait()
            di1.wait()
            di2.wait()
            return 0

        lax.fori_loop(0, NB, edge_step, 0)
        plsc.subcore_barrier()

        def write_step(k, _):
            base = s * PT + k * W
            gb = coff + base
            sv = rvN.at[pl.ds(0, W), :]
            pltpu.sync_copy(acc_sh.at[pl.ds(base, W), :], sv)
            pltpu.sync_copy(deg_sh.at[pl.ds(base, W)], dvv)
            pltpu.sync_copy(macc.at[pl.ds(gb, W), :], wv)

            def out_step(g, _):
                d16 = dvv[pl.ds(g * 16, 16)]
                for i in range(16):
                    n = g * 16 + i
                    d = jnp.full((16,), d16[i])
                    e0 = d * sv[n, pl.ds(0, 16)]
                    e1 = d * sv[n, pl.ds(16, 16)]
                    m0 = wv[n, pl.ds(0, 16)] + e0
                    m1 = wv[n, pl.ds(16, 16)] + e1
                    if last:
                        wv[n, pl.ds(0, 16)] = m0 * 0.25
                        wv[n, pl.ds(16, 16)] = m1 * 0.25
                    else:
                        wv[n, pl.ds(0, 16)] = m0
                        wv[n, pl.ds(16, 16)] = m1
                        sv[n, pl.ds(0, 16)] = d * e0
                        sv[n, pl.ds(16, 16)] = d * e1
                return 0

            lax.fori_loop(0, W // 16, out_step, 0)
            if last:
                pltpu.sync_copy(wv, outf.at[pl.ds(gb, W), :])
            else:
                pltpu.sync_copy(wv, macc.at[pl.ds(gb, W), :])
                pltpu.sync_copy(sv, gsrc.at[pl.ds(gb, W), :])
            return 0

        lax.fori_loop(0, NK, write_step, 0)
        plsc.subcore_barrier()


_mesh = plsc.VectorSubcoreMesh(core_axis_name="c", subcore_axis_name="s")

_sc_call = pl.kernel(
    _body,
    out_type=(
        jax.ShapeDtypeStruct((2 * NN, 32), jnp.float32),
        jax.ShapeDtypeStruct((2 * NN, 32), jnp.float32),
        jax.ShapeDtypeStruct((2 * NN, 32), jnp.float32),
    ),
    mesh=_mesh,
    compiler_params=pltpu.CompilerParams(use_tc_tiling_on_sc=False),
    scratch_types=[
        pltpu.VMEM_SHARED((NN, 32), jnp.float32),
        pltpu.VMEM_SHARED((NN,), jnp.float32),
        pltpu.VMEM((W, 32), jnp.float32),
        pltpu.VMEM((F * ECH, 32), jnp.float32),
        pltpu.VMEM((W,), jnp.float32),
        pltpu.VMEM((2 * F, ECH), jnp.int32),
        pltpu.VMEM((2 * F, ECH), jnp.int32),
        pltpu.VMEM((1, ECH), jnp.float32),
        pltpu.SemaphoreType.DMA,
        pltpu.SemaphoreType.DMA,
        pltpu.SemaphoreType.DMA,
    ],
)


@jax.jit
def kernel(edge_index, embedding_weight):
    row = edge_index[0]
    col = edge_index[1]
    pad = jnp.full((EPAD - E,), TRASH, jnp.int32)
    rowp = jnp.concatenate([row, pad]).reshape(16, NB, F, ECH)
    rowp = jnp.concatenate(
        [rowp, jnp.zeros((16, 1, F, ECH), jnp.int32)], axis=1)
    colp0 = jnp.concatenate([col, pad]).reshape(16, NB, F, ECH)
    colp0 = jnp.concatenate(
        [colp0, jnp.zeros((16, 1, F, ECH), jnp.int32)], axis=1)
    colp = jnp.stack([colp0, colp0 + NN])
    embp = (jnp.zeros((2 * NN, 32), jnp.float32)
            .at[:NV].set(embedding_weight[:, :32])
            .at[NN:NN + NV].set(embedding_weight[:, 32:]))
    z2 = jnp.zeros((W, 32), jnp.float32)
    z1 = jnp.zeros((PT,), jnp.float32)
    outf, _, _ = _sc_call(rowp, colp, embp, z2, z1)
    final = jnp.concatenate([outf[:NV], outf[NN:NN + NV]], axis=1)
    return final[:NV // 2], final[NV // 2:]
